# baseline (speedup 1.0000x reference)
import functools

import jax
import jax.numpy as jnp
from jax import lax
from jax.experimental import pallas as pl
from jax.experimental.pallas import tpu as pltpu
from jax.experimental.pallas import tpu_sc as plsc

_B = 512
_NPIX = 3 * 224 * 224
_NRET = 128
_NLGN = 128
_NV1 = 256
_NIT = 128
_NCLS = 1000
_ROWS = _NPIX // 16
_NC, _NS, _L = 2, 16, 16
_NW = _NC * _NS
_BPW = _B // _NW
_NB = _NRET // _L


def _sigmoid(z):
    return 1.0 / (1.0 + jnp.exp(-z))


def _retina_sc(xv, pm_flat, wrt, bret):
    mesh = plsc.VectorSubcoreMesh(core_axis_name="c", subcore_axis_name="s")

    @functools.partial(
        pl.kernel,
        out_type=jax.ShapeDtypeStruct((_B, _NRET), jnp.float32),
        mesh=mesh,
        scratch_types=[
            pltpu.VMEM((_NRET * 16,), jnp.int32),
            pltpu.VMEM((_L, _NRET), jnp.float32),
            pltpu.VMEM((_NRET,), jnp.float32),
            pltpu.VMEM((2 * _NRET,), jnp.int32),
            pltpu.VMEM((_NRET,), jnp.int32),
            pltpu.VMEM((2 * _BPW, 2 * _NRET // 2), jnp.int32),
            pltpu.VMEM((2 * _NRET * _BPW, _L), jnp.float32),
            pltpu.VMEM((_BPW, _NRET), jnp.float32),
            pltpu.SemaphoreType.DMA,
        ],
        compiler_params=pltpu.CompilerParams(needs_layout_passes=False,
                                             use_tc_tiling_on_sc=False),
    )
    def k(xv_h, pm_h, wrt_h, bret_h, out_h,
          pm_v, wrt_v, bret_v, biv, rem_v, iv, stage, r1l, sem):
        cid = lax.axis_index("c")
        sid = lax.axis_index("s")
        wid = sid * _NC + cid
        b0 = wid * _BPW
        iota = lax.iota(jnp.int32, _L)

        pltpu.sync_copy(pm_h, pm_v)
        pltpu.sync_copy(wrt_h, wrt_v)
        pltpu.sync_copy(bret_h, bret_v)

        for nb in range(_NB):
            n = nb * _L + iota
            sv = plsc.load_gather(pm_v, [n * 16])
            s16 = lax.shift_right_logical(sv, 4)
            rem = lax.bitwise_and(sv, 15)
            rem_v[pl.ds(nb * _L, _L)] = rem
            plsc.store_scatter(biv, [n * 2], s16)
            plsc.store_scatter(biv, [n * 2 + 1],
                               jnp.minimum(s16 + 1, _ROWS - 1))

        def build_iv(i, carry):
            b = lax.shift_right_logical(i, 1)
            h = lax.bitwise_and(i, 1)
            off = (b0 + b) * _ROWS
            for c in range(8):
                base = plsc.load_gather(biv, [h * 128 + c * _L + iota])
                plsc.store_scatter(iv, [jnp.broadcast_to(i, (_L,)),
                                        c * _L + iota], base + off)
            return carry
        lax.fori_loop(0, 2 * _BPW, build_iv, 0)

        def fire(i, carry):
            pltpu.async_copy(xv_h.at[iv.at[i]],
                             stage.at[pl.ds(i * 128, 128)], sem)
            return carry
        lax.fori_loop(0, 2 * _BPW, fire, 0)

        def drain(i, carry):
            pltpu.make_async_copy(xv_h.at[iv.at[i]],
                                  stage.at[pl.ds(i * 128, 128)], sem).wait()
            return carry
        lax.fori_loop(0, 2 * _BPW, drain, 0)

        def compute(b, carry):
            for nb in range(_NB):
                rem = rem_v[pl.ds(nb * _L, _L)]
                base = b * 4096 + (nb * _L + iota) * 32 + rem
                acc = bret_v[pl.ds(nb * _L, _L)]
                for j in range(16):
                    flat = base + j
                    ri = lax.shift_right_logical(flat, 4)
                    ci = lax.bitwise_and(flat, 15)
                    xj = plsc.load_gather(stage, [ri, ci])
                    acc = acc + xj * wrt_v[j, pl.ds(nb * _L, _L)]
                r1 = _sigmoid(acc)
                plsc.store_scatter(r1l, [jnp.broadcast_to(b, (_L,)),
                                         nb * _L + iota], r1)
            return carry
        lax.fori_loop(0, _BPW, compute, 0)

        pltpu.sync_copy(r1l, out_h.at[pl.ds(b0, _BPW)])

    return k(xv, pm_flat, wrt, bret)


def _dense_tc_body(r1_ref, wl_ref, bl_ref, m1_ref, wv_ref, bv_ref,
                   m2_ref, wi_ref, bi_ref, wc_ref, bc_ref, out_ref):
    f32 = jnp.float32
    hi = lax.Precision.HIGHEST
    r1 = r1_ref[...]
    wl = jnp.sum(wl_ref[...], axis=1)
    r2 = _sigmoid(r1 * wl[None, :] + bl_ref[...])
    m1 = m1_ref[...]
    deg1 = jnp.sum(m1, axis=1)
    mv1 = lax.dot_general(r2, m1, (((1,), (1,)), ((), ())),
                          precision=hi, preferred_element_type=f32)
    mv1 = mv1 / deg1[None, :]
    r3 = _sigmoid(mv1 * jnp.sum(wv_ref[...], axis=1)[None, :] + bv_ref[...])
    m2 = m2_ref[...]
    deg2 = jnp.sum(m2, axis=1)
    mit = lax.dot_general(r3, m2, (((1,), (1,)), ((), ())),
                          precision=hi, preferred_element_type=f32)
    mit = mit / deg2[None, :]
    r4 = _sigmoid(mit * jnp.sum(wi_ref[...], axis=1)[None, :] + bi_ref[...])
    out_ref[...] = lax.dot_general(r4, wc_ref[...], (((1,), (1,)), ((), ())),
                                   precision=hi,
                                   preferred_element_type=f32) + bc_ref[...]


def _dense_tc(r1, W_lgn2, b_lgn2, m1, W_v12, b_v12, m2, W_it2, b_it2,
              W_cls, b_cls2):
    return pl.pallas_call(
        _dense_tc_body,
        out_shape=jax.ShapeDtypeStruct((_B, _NCLS), jnp.float32),
    )(r1, W_lgn2, b_lgn2, m1, W_v12, b_v12, m2, W_it2, b_it2, W_cls, b_cls2)


def kernel(x, W_ret, b_ret, W_lgn, b_lgn, W_v1, b_v1, W_it, b_it,
           W_cls, b_cls, pixel_map, lgn_to_v1, v1_to_it):
    xv = x.reshape(_B * _ROWS, 16)
    pm_flat = pixel_map.reshape(-1).astype(jnp.int32)
    wrt = W_ret.reshape(_NRET, 16).T
    r1 = _retina_sc(xv, pm_flat, wrt, b_ret)
    logits = _dense_tc(
        r1,
        W_lgn.reshape(_NLGN, 16), b_lgn.reshape(1, _NLGN),
        lgn_to_v1, W_v1.reshape(_NV1, 32), b_v1.reshape(1, _NV1),
        v1_to_it, W_it.reshape(_NIT, 32), b_it.reshape(1, _NIT),
        W_cls, b_cls.reshape(1, _NCLS),
    )
    return logits


# speedup vs baseline: 15.8929x; 15.8929x over previous
import functools

import jax
import jax.numpy as jnp
from jax import lax
from jax.experimental import pallas as pl
from jax.experimental.pallas import tpu as pltpu
from jax.experimental.pallas import tpu_sc as plsc

_B = 512
_NPIX = 3 * 224 * 224
_NRET = 128
_NLGN = 128
_NV1 = 256
_NIT = 128
_NCLS = 1000
_NC, _NS, _L = 2, 16, 16
_NW = _NC * _NS
_NPB = 8
_HB = _B // 2


def _sigmoid(z):
    return 1.0 / (1.0 + jnp.exp(-z))


def _retina_sc(xt, pm_flat, wr, bret):
    mesh = plsc.VectorSubcoreMesh(core_axis_name="c", subcore_axis_name="s")

    @functools.partial(
        pl.kernel,
        out_type=jax.ShapeDtypeStruct((_NRET, _B), jnp.float32),
        mesh=mesh,
        scratch_types=[
            pltpu.VMEM((_NPB * 16,), jnp.int32),
            pltpu.VMEM((_NRET, 16), jnp.float32),
            pltpu.VMEM((_NRET,), jnp.float32),
            pltpu.VMEM((_NPB * 16, _B), jnp.float32),
            pltpu.VMEM((_NPB, _B), jnp.float32),
            pltpu.SemaphoreType.DMA,
        ],
        compiler_params=pltpu.CompilerParams(needs_layout_passes=False),
    )
    def k(xt_h, pm_h, wr_h, bret_h, out_h, iv, wr_v, bret_v, stage, r1l, sem):
        cid = lax.axis_index("c")
        sid = lax.axis_index("s")
        wid = sid * _NC + cid
        nb = lax.shift_right_logical(wid, 1)
        h = lax.bitwise_and(wid, 1)
        n0 = nb * _NPB
        col0 = h * _HB
        iota = lax.iota(jnp.int32, _L)

        pltpu.sync_copy(pm_h.at[pl.ds(n0 * 16, _NPB * 16)], iv)
        pltpu.sync_copy(wr_h, wr_v)
        pltpu.sync_copy(bret_h, bret_v)

        pltpu.async_copy(xt_h.at[iv], stage, sem)
        pltpu.make_async_copy(xt_h.at[iv], stage, sem).wait()

        def compute(bb, carry):
            col = col0 + bb * _L
            cols = col + iota
            for nl in range(_NPB):
                acc = plsc.load_gather(bret_v, [jnp.broadcast_to(n0 + nl, (_L,))])
                for j in range(16):
                    row = nl * 16 + j
                    xv = plsc.load_gather(stage,
                                          [jnp.broadcast_to(row, (_L,)), cols])
                    wv = plsc.load_gather(
                        wr_v, [jnp.broadcast_to(n0 + nl, (_L,)),
                               jnp.broadcast_to(j, (_L,))])
                    acc = acc + xv * wv
                r1 = _sigmoid(acc)
                plsc.store_scatter(r1l, [jnp.broadcast_to(nl, (_L,)), cols], r1)
            return carry
        lax.fori_loop(0, _HB // _L, compute, 0)

        pltpu.sync_copy(r1l.at[:, pl.ds(col0, _HB)],
                        out_h.at[pl.ds(n0, _NPB), pl.ds(col0, _HB)])

    return k(xt, pm_flat, wr, bret)


def _dense_tc_body(r1_ref, wl_ref, bl_ref, m1_ref, wv_ref, bv_ref,
                   m2_ref, wi_ref, bi_ref, wc_ref, bc_ref, out_ref):
    f32 = jnp.float32
    hi = lax.Precision.HIGHEST
    r1t = r1_ref[...]
    wl = jnp.sum(wl_ref[...], axis=1)
    r2t = _sigmoid(r1t * wl[:, None] + bl_ref[...])
    m1 = m1_ref[...]
    deg1 = jnp.sum(m1, axis=1)
    mv1t = lax.dot_general(m1, r2t, (((1,), (0,)), ((), ())),
                           precision=hi, preferred_element_type=f32)
    mv1t = mv1t / deg1[:, None]
    r3t = _sigmoid(mv1t * jnp.sum(wv_ref[...], axis=1)[:, None] + bv_ref[...])
    m2 = m2_ref[...]
    deg2 = jnp.sum(m2, axis=1)
    mitt = lax.dot_general(m2, r3t, (((1,), (0,)), ((), ())),
                           precision=hi, preferred_element_type=f32)
    mitt = mitt / deg2[:, None]
    r4t = _sigmoid(mitt * jnp.sum(wi_ref[...], axis=1)[:, None] + bi_ref[...])
    out_ref[...] = lax.dot_general(wc_ref[...], r4t, (((1,), (0,)), ((), ())),
                                   precision=hi,
                                   preferred_element_type=f32) + bc_ref[...]


def _dense_tc(r1t, W_lgn2, b_lgn2, m1, W_v12, b_v12, m2, W_it2, b_it2,
              W_cls, b_cls2):
    return pl.pallas_call(
        _dense_tc_body,
        out_shape=jax.ShapeDtypeStruct((_NCLS, _B), jnp.float32),
    )(r1t, W_lgn2, b_lgn2, m1, W_v12, b_v12, m2, W_it2, b_it2, W_cls, b_cls2)


def kernel(x, W_ret, b_ret, W_lgn, b_lgn, W_v1, b_v1, W_it, b_it,
           W_cls, b_cls, pixel_map, lgn_to_v1, v1_to_it):
    xt = x.reshape(_B, _NPIX).T
    pm_flat = pixel_map.reshape(-1).astype(jnp.int32)
    wr = W_ret.reshape(_NRET, 16)
    r1t = _retina_sc(xt, pm_flat, wr, b_ret)
    logits_t = _dense_tc(
        r1t,
        W_lgn.reshape(_NLGN, 16), b_lgn.reshape(_NLGN, 1),
        lgn_to_v1, W_v1.reshape(_NV1, 32), b_v1.reshape(_NV1, 1),
        v1_to_it, W_it.reshape(_NIT, 32), b_it.reshape(_NIT, 1),
        W_cls, b_cls.reshape(_NCLS, 1),
    )
    return logits_t.T


# speedup vs baseline: 18.0026x; 1.1327x over previous
import functools

import jax
import jax.numpy as jnp
from jax import lax
from jax.experimental import pallas as pl
from jax.experimental.pallas import tpu as pltpu
from jax.experimental.pallas import tpu_sc as plsc

_B = 512
_NPIX = 3 * 224 * 224
_NRET = 128
_NLGN = 128
_NV1 = 256
_NIT = 128
_NCLS = 1000
_NC, _NS, _L = 2, 16, 16
_NW = _NC * _NS
_NPB = 8
_HB = _B // 2


def _sigmoid(z):
    return 1.0 / (1.0 + jnp.exp(-z))


_NG = 16
_QB = 128


def _retina_sc(xt, pm_flat, wr, bret):
    mesh = plsc.VectorSubcoreMesh(core_axis_name="c", subcore_axis_name="s")

    @functools.partial(
        pl.kernel,
        out_type=jax.ShapeDtypeStruct((_NRET, _B), jnp.float32),
        mesh=mesh,
        scratch_types=[
            pltpu.VMEM((2 * _QB,), jnp.int32),
            pltpu.VMEM((_NRET, 16), jnp.float32),
            pltpu.VMEM((_NRET,), jnp.float32),
            pltpu.VMEM((_NG * 16, _QB), jnp.float32),
            pltpu.VMEM((_NG, _QB), jnp.float32),
            pltpu.SemaphoreType.DMA,
        ],
        compiler_params=pltpu.CompilerParams(needs_layout_passes=False),
    )
    def k(xt_h, pm_h, wr_h, bret_h, out_h, iv, wr_v, bret_v, stage, r1l, sem):
        cid = lax.axis_index("c")
        sid = lax.axis_index("s")
        wid = sid * _NC + cid
        ng = lax.shift_right_logical(wid, 2)
        q = lax.bitwise_and(wid, 3)
        n0 = pl.multiple_of(ng * _NG, _NG)
        col0 = pl.multiple_of(q * _QB, _QB)

        pltpu.sync_copy(pm_h.at[pl.ds(n0 * 16, 2 * _QB)], iv)
        pltpu.sync_copy(wr_h, wr_v)
        pltpu.sync_copy(bret_h, bret_v)

        for g in range(2):
            pltpu.async_copy(
                xt_h.at[iv.at[pl.ds(g * _QB, _QB)], pl.ds(col0, _QB)],
                stage.at[pl.ds(g * _QB, _QB)], sem)
        for g in range(2):
            pltpu.make_async_copy(
                xt_h.at[iv.at[pl.ds(g * _QB, _QB)], pl.ds(col0, _QB)],
                stage.at[pl.ds(g * _QB, _QB)], sem).wait()

        def compute(nl, carry):
            n = n0 + nl
            nsp = jnp.broadcast_to(n, (_L,))
            acc0 = plsc.load_gather(bret_v, [nsp])
            wsp = [plsc.load_gather(wr_v, [nsp, jnp.broadcast_to(j, (_L,))])
                   for j in range(16)]
            for bb in range(_QB // _L):
                acc = acc0
                for j in range(16):
                    acc = acc + stage[nl * 16 + j, pl.ds(bb * _L, _L)] * wsp[j]
                r1l[nl, pl.ds(bb * _L, _L)] = _sigmoid(acc)
            return carry
        lax.fori_loop(0, _NG, compute, 0)

        pltpu.sync_copy(r1l, out_h.at[pl.ds(n0, _NG), pl.ds(col0, _QB)])

    return k(xt, pm_flat, wr, bret)


def _dense_tc_body(r1_ref, wl_ref, bl_ref, m1_ref, wv_ref, bv_ref,
                   m2_ref, wi_ref, bi_ref, wc_ref, bc_ref, out_ref):
    f32 = jnp.float32
    hi = lax.Precision.HIGHEST
    r1t = r1_ref[...]
    wl = jnp.sum(wl_ref[...], axis=1)
    r2t = _sigmoid(r1t * wl[:, None] + bl_ref[...])
    m1 = m1_ref[...]
    deg1 = jnp.sum(m1, axis=1)
    mv1t = lax.dot_general(m1, r2t, (((1,), (0,)), ((), ())),
                           precision=hi, preferred_element_type=f32)
    mv1t = mv1t / deg1[:, None]
    r3t = _sigmoid(mv1t * jnp.sum(wv_ref[...], axis=1)[:, None] + bv_ref[...])
    m2 = m2_ref[...]
    deg2 = jnp.sum(m2, axis=1)
    mitt = lax.dot_general(m2, r3t, (((1,), (0,)), ((), ())),
                           precision=hi, preferred_element_type=f32)
    mitt = mitt / deg2[:, None]
    r4t = _sigmoid(mitt * jnp.sum(wi_ref[...], axis=1)[:, None] + bi_ref[...])
    out_ref[...] = lax.dot_general(wc_ref[...], r4t, (((1,), (0,)), ((), ())),
                                   precision=hi,
                                   preferred_element_type=f32) + bc_ref[...]


def _dense_tc(r1t, W_lgn2, b_lgn2, m1, W_v12, b_v12, m2, W_it2, b_it2,
              W_cls, b_cls2):
    return pl.pallas_call(
        _dense_tc_body,
        out_shape=jax.ShapeDtypeStruct((_NCLS, _B), jnp.float32),
    )(r1t, W_lgn2, b_lgn2, m1, W_v12, b_v12, m2, W_it2, b_it2, W_cls, b_cls2)


def kernel(x, W_ret, b_ret, W_lgn, b_lgn, W_v1, b_v1, W_it, b_it,
           W_cls, b_cls, pixel_map, lgn_to_v1, v1_to_it):
    xt = x.reshape(_B, _NPIX).T
    pm_flat = pixel_map.reshape(-1).astype(jnp.int32)
    wr = W_ret.reshape(_NRET, 16)
    r1t = _retina_sc(xt, pm_flat, wr, b_ret)
    logits_t = _dense_tc(
        r1t,
        W_lgn.reshape(_NLGN, 16), b_lgn.reshape(_NLGN, 1),
        lgn_to_v1, W_v1.reshape(_NV1, 32), b_v1.reshape(_NV1, 1),
        v1_to_it, W_it.reshape(_NIT, 32), b_it.reshape(_NIT, 1),
        W_cls, b_cls.reshape(_NCLS, 1),
    )
    return logits_t.T
